# baseline (speedup 1.0000x reference)
import functools

import jax
import jax.numpy as jnp
from jax import lax
from jax.experimental import pallas as pl
from jax.experimental.pallas import tpu as pltpu, tpu_sc as plsc

B, S = 128, 512
V, H, P, T = 30522, 768, 512, 2
EPS = 1e-12

NW = 32
TOK = B * S
K = 4
TOKS = TOK // K
BS = B // K
TPW = TOKS // NW
C = 128
NC = TPW // C
H2 = H // 2


def _gather_body(ids_hbm, word_hbm, out_hbm, idx_all, buf_a, buf_b,
                 gsem_a, gsem_b, osem_a, osem_b):
    wid = lax.axis_index("s") * 2 + lax.axis_index("c")
    tok0 = wid * TPW
    pltpu.sync_copy(ids_hbm.at[pl.ds(tok0, TPW)], idx_all)

    def gstart(i, buf, sem):
        pltpu.async_copy(word_hbm.at[idx_all.at[pl.ds(i * C, C)]], buf, sem)

    def gwait(i, buf, sem):
        pltpu.make_async_copy(
            word_hbm.at[idx_all.at[pl.ds(i * C, C)]], buf, sem).wait()

    def ostart(i, buf, sem):
        return pltpu.async_copy(
            buf, out_hbm.at[pl.ds(tok0 + i * C, C)], sem)

    gstart(0, buf_a, gsem_a)
    gstart(1, buf_b, gsem_b)

    def pair_body(p, carry):
        i = 2 * p
        gwait(i, buf_a, gsem_a)
        oa = ostart(i, buf_a, osem_a)
        gwait(i + 1, buf_b, gsem_b)
        ob = ostart(i + 1, buf_b, osem_b)
        oa.wait()

        @pl.when(i + 2 < NC)
        def _():
            gstart(i + 2, buf_a, gsem_a)

        ob.wait()

        @pl.when(i + 3 < NC)
        def _():
            gstart(i + 3, buf_b, gsem_b)

        return carry

    lax.fori_loop(0, NC // 2, pair_body, 0)


def _sc_gather(ids_flat, word_emb):
    mesh = plsc.VectorSubcoreMesh(core_axis_name="c", subcore_axis_name="s")
    f = pl.kernel(
        _gather_body,
        out_type=jax.ShapeDtypeStruct((TOKS, H2), jnp.int32),
        mesh=mesh,
        scratch_types=[
            pltpu.VMEM((TPW,), jnp.int32),
            pltpu.VMEM((C, H2), jnp.int32),
            pltpu.VMEM((C, H2), jnp.int32),
            pltpu.SemaphoreType.DMA,
            pltpu.SemaphoreType.DMA,
            pltpu.SemaphoreType.DMA,
            pltpu.SemaphoreType.DMA,
        ],
        compiler_params=pltpu.CompilerParams(needs_layout_passes=False),
    )
    return f(ids_flat, word_emb)


GB = 8


def _ln_kernel(tt_ref, x_ref, pos_ref, typ_ref, gamma_ref, beta_ref,
               *maybe_acc_and_o):
    o_ref = maybe_acc_and_o[-1]
    xi = x_ref[...]
    lo = lax.bitcast_convert_type(
        lax.shift_left(xi, 16), jnp.float32)
    hi = lax.bitcast_convert_type(
        jnp.bitwise_and(xi, jnp.int32(-65536)), jnp.float32)
    x = jnp.concatenate([lo, hi], axis=-1)
    ttf = tt_ref[:, 0, :]
    t0 = typ_ref[0, :]
    tdiff = typ_ref[1, :] - typ_ref[0, :]
    v = x + pos_ref[...] + t0[None, None, :] \
        + ttf[:, :, None] * tdiff[None, None, :]
    mean = jnp.mean(v, axis=-1, keepdims=True)
    var = jnp.mean(v * v, axis=-1, keepdims=True) - mean * mean
    inv = lax.rsqrt(var + EPS)
    o_ref[...] = ((v - mean) * inv) * gamma_ref[0, :][None, None, :] \
        + beta_ref[0, :][None, None, :]


def _tc_ln(k, ttf2d, rows, pos_emb, type_emb, gamma, beta, acc=None):
    goff = k * (BS // GB)
    in_specs = [
        pl.BlockSpec((GB, 1, S), lambda i: (i, 0, 0)),
        pl.BlockSpec((GB, S, H2), lambda i: (i, 0, 0)),
        pl.BlockSpec((1, S, H), lambda i: (0, 0, 0)),
        pl.BlockSpec((T, H), lambda i: (0, 0)),
        pl.BlockSpec((1, H), lambda i: (0, 0)),
        pl.BlockSpec((1, H), lambda i: (0, 0)),
    ]
    args = [ttf2d, rows, pos_emb, type_emb, gamma, beta]
    aliases = {}
    if acc is not None:
        in_specs.append(pl.BlockSpec(memory_space=pl.ANY))
        args.append(acc)
        aliases = {6: 0}
    return pl.pallas_call(
        _ln_kernel,
        grid=(BS // GB,),
        in_specs=in_specs,
        out_specs=pl.BlockSpec((GB, S, H), lambda i: (goff + i, 0, 0)),
        out_shape=jax.ShapeDtypeStruct((B, S, H), jnp.float32),
        input_output_aliases=aliases,
    )(*args)


@jax.jit
def _run(ids_flat, ttf2d, word_emb, pos_emb, type_emb, gamma2d, beta2d):
    def rne16(f):
        i = lax.bitcast_convert_type(f, jnp.int32)
        r = i + 0x7FFF + jnp.bitwise_and(lax.shift_right_logical(i, 16), 1)
        return lax.shift_right_logical(r, 16)

    wa = rne16(word_emb[:, :H2])
    wb = rne16(word_emb[:, H2:])
    word_i32 = jnp.bitwise_or(wa, lax.shift_left(wb, 16)).astype(jnp.int32)
    rows = [_sc_gather(ids_flat[k * TOKS:(k + 1) * TOKS], word_i32)
            for k in range(K)]
    pos3 = pos_emb.reshape(1, S, H)
    acc = None
    for k in range(K):
        acc = _tc_ln(k, ttf2d[k * BS:(k + 1) * BS],
                     rows[k].reshape(BS, S, H2), pos3, type_emb, gamma2d,
                     beta2d, acc)
    return acc


def kernel(input_ids, token_type_ids, word_emb, pos_emb, type_emb, gamma,
           beta):
    ids_flat = input_ids.astype(jnp.int32).reshape(TOK)
    ttf2d = token_type_ids.astype(jnp.float32).reshape(B, 1, S)
    return _run(ids_flat, ttf2d, word_emb, pos_emb, type_emb,
                gamma.reshape(1, H), beta.reshape(1, H))


# speedup vs baseline: 1.0241x; 1.0241x over previous
import functools

import jax
import jax.numpy as jnp
from jax import lax
from jax.experimental import pallas as pl
from jax.experimental.pallas import tpu as pltpu, tpu_sc as plsc

B, S = 128, 512
V, H, P, T = 30522, 768, 512, 2
EPS = 1e-12

NW = 32
TOK = B * S
K = 2
TOKS = TOK // K
BS = B // K
TPW = TOKS // NW
C = 128
NC = TPW // C
H2 = H // 2


def _gather_body(ids_hbm, word_hbm, out_hbm, idx_all, buf_a, buf_b,
                 gsem_a, gsem_b, osem_a, osem_b):
    wid = lax.axis_index("s") * 2 + lax.axis_index("c")
    tok0 = wid * TPW
    pltpu.sync_copy(ids_hbm.at[pl.ds(tok0, TPW)], idx_all)

    def gstart(i, buf, sem):
        pltpu.async_copy(word_hbm.at[idx_all.at[pl.ds(i * C, C)]], buf, sem)

    def gwait(i, buf, sem):
        pltpu.make_async_copy(
            word_hbm.at[idx_all.at[pl.ds(i * C, C)]], buf, sem).wait()

    def ostart(i, buf, sem):
        return pltpu.async_copy(
            buf, out_hbm.at[pl.ds(tok0 + i * C, C)], sem)

    gstart(0, buf_a, gsem_a)
    gstart(1, buf_b, gsem_b)

    def pair_body(p, carry):
        i = 2 * p
        gwait(i, buf_a, gsem_a)
        oa = ostart(i, buf_a, osem_a)
        gwait(i + 1, buf_b, gsem_b)
        ob = ostart(i + 1, buf_b, osem_b)
        oa.wait()

        @pl.when(i + 2 < NC)
        def _():
            gstart(i + 2, buf_a, gsem_a)

        ob.wait()

        @pl.when(i + 3 < NC)
        def _():
            gstart(i + 3, buf_b, gsem_b)

        return carry

    lax.fori_loop(0, NC // 2, pair_body, 0)


def _sc_gather(ids_flat, word_emb):
    mesh = plsc.VectorSubcoreMesh(core_axis_name="c", subcore_axis_name="s")
    f = pl.kernel(
        _gather_body,
        out_type=jax.ShapeDtypeStruct((TOKS, H2), jnp.int32),
        mesh=mesh,
        scratch_types=[
            pltpu.VMEM((TPW,), jnp.int32),
            pltpu.VMEM((C, H2), jnp.int32),
            pltpu.VMEM((C, H2), jnp.int32),
            pltpu.SemaphoreType.DMA,
            pltpu.SemaphoreType.DMA,
            pltpu.SemaphoreType.DMA,
            pltpu.SemaphoreType.DMA,
        ],
        compiler_params=pltpu.CompilerParams(needs_layout_passes=False),
    )
    return f(ids_flat, word_emb)


GB = 8


def _ln_kernel(tt_ref, x_ref, pos_ref, typ_ref, gamma_ref, beta_ref,
               *maybe_acc_and_o):
    o_ref = maybe_acc_and_o[-1]
    xi = x_ref[...]
    lo = lax.bitcast_convert_type(
        lax.shift_left(xi, 16), jnp.float32)
    hi = lax.bitcast_convert_type(
        jnp.bitwise_and(xi, jnp.int32(-65536)), jnp.float32)
    x = jnp.concatenate([lo, hi], axis=-1)
    ttf = tt_ref[:, 0, :]
    t0 = typ_ref[0, :]
    tdiff = typ_ref[1, :] - typ_ref[0, :]
    v = x + pos_ref[...] + t0[None, None, :] \
        + ttf[:, :, None] * tdiff[None, None, :]
    mean = jnp.mean(v, axis=-1, keepdims=True)
    var = jnp.mean(v * v, axis=-1, keepdims=True) - mean * mean
    inv = lax.rsqrt(var + EPS)
    o_ref[...] = ((v - mean) * inv) * gamma_ref[0, :][None, None, :] \
        + beta_ref[0, :][None, None, :]


def _tc_ln(k, ttf2d, rows, pos_emb, type_emb, gamma, beta, acc=None):
    goff = k * (BS // GB)
    in_specs = [
        pl.BlockSpec((GB, 1, S), lambda i: (i, 0, 0)),
        pl.BlockSpec((GB, S, H2), lambda i: (i, 0, 0)),
        pl.BlockSpec((1, S, H), lambda i: (0, 0, 0)),
        pl.BlockSpec((T, H), lambda i: (0, 0)),
        pl.BlockSpec((1, H), lambda i: (0, 0)),
        pl.BlockSpec((1, H), lambda i: (0, 0)),
    ]
    args = [ttf2d, rows, pos_emb, type_emb, gamma, beta]
    aliases = {}
    if acc is not None:
        in_specs.append(pl.BlockSpec(memory_space=pl.ANY))
        args.append(acc)
        aliases = {6: 0}
    return pl.pallas_call(
        _ln_kernel,
        grid=(BS // GB,),
        in_specs=in_specs,
        out_specs=pl.BlockSpec((GB, S, H), lambda i: (goff + i, 0, 0)),
        out_shape=jax.ShapeDtypeStruct((B, S, H), jnp.float32),
        input_output_aliases=aliases,
    )(*args)


@jax.jit
def _run(ids_flat, ttf2d, word_emb, pos_emb, type_emb, gamma2d, beta2d):
    def rne16(f):
        i = lax.bitcast_convert_type(f, jnp.int32)
        r = i + 0x7FFF + jnp.bitwise_and(lax.shift_right_logical(i, 16), 1)
        return lax.shift_right_logical(r, 16)

    wa = rne16(word_emb[:, :H2])
    wb = rne16(word_emb[:, H2:])
    word_i32 = jnp.bitwise_or(wa, lax.shift_left(wb, 16)).astype(jnp.int32)
    rows = [_sc_gather(ids_flat[k * TOKS:(k + 1) * TOKS], word_i32)
            for k in range(K)]
    pos3 = pos_emb.reshape(1, S, H)
    acc = None
    for k in range(K):
        acc = _tc_ln(k, ttf2d[k * BS:(k + 1) * BS],
                     rows[k].reshape(BS, S, H2), pos3, type_emb, gamma2d,
                     beta2d, acc)
    return acc


def kernel(input_ids, token_type_ids, word_emb, pos_emb, type_emb, gamma,
           beta):
    ids_flat = input_ids.astype(jnp.int32).reshape(TOK)
    ttf2d = token_type_ids.astype(jnp.float32).reshape(B, 1, S)
    return _run(ids_flat, ttf2d, word_emb, pos_emb, type_emb,
                gamma.reshape(1, H), beta.reshape(1, H))
